# baseline (speedup 1.0000x reference)
import functools

import jax
import jax.numpy as jnp
from jax import lax
from jax.experimental import pallas as pl
from jax.experimental.pallas import tpu as pltpu
from jax.experimental.pallas import tpu_sc as plsc

N = 10000
D = 128
R_ENC = 8
E_PER = 40000
L = 2
B = 8192
EPS = 1e-5

NW = 32
NS = 16
XC = D // NW
E_PAD = 40960
ECH = 2048
NCH = E_PAD // ECH
NP = 10240
ROW_BLK = 2048


def _lnT_body(xT_ref, s_ref, sw_ref, wcat_ref, g_ref, b_ref, o_ref, *, relu):
    selfT = jnp.dot(sw_ref[...], xT_ref[...],
                    preferred_element_type=jnp.float32)
    msgT = jnp.dot(wcat_ref[...], s_ref[...],
                   preferred_element_type=jnp.float32)
    tot = selfT + msgT
    mu = jnp.mean(tot, axis=0, keepdims=True)
    var = jnp.mean((tot - mu) * (tot - mu), axis=0, keepdims=True)
    hh = (tot - mu) * lax.rsqrt(var + EPS) * g_ref[...] + b_ref[...]
    if relu:
        hh = jnp.maximum(hh, 0.0)
    o_ref[...] = xT_ref[...] + hh


def _tc_layerT(xT, s_cat, sw, wcat, g, b, relu):
    return pl.pallas_call(
        functools.partial(_lnT_body, relu=relu),
        grid=(NP // ROW_BLK,),
        in_specs=[
            pl.BlockSpec((D, ROW_BLK), lambda i: (0, i)),
            pl.BlockSpec((R_ENC * D, ROW_BLK), lambda i: (0, i)),
            pl.BlockSpec((D, D), lambda i: (0, 0)),
            pl.BlockSpec((D, R_ENC * D), lambda i: (0, 0)),
            pl.BlockSpec((D, 1), lambda i: (0, 0)),
            pl.BlockSpec((D, 1), lambda i: (0, 0)),
        ],
        out_specs=pl.BlockSpec((D, ROW_BLK), lambda i: (0, i)),
        out_shape=jax.ShapeDtypeStruct((D, NP), jnp.float32),
    )(xT, s_cat, sw, wcat, g, b)


def _score_body(a_ref, b_ref, c_ref, o_ref):
    o_ref[...] = jnp.sum(a_ref[...] * b_ref[...] * c_ref[...],
                         axis=-1, keepdims=True)


def _tc_score(xh, rd, xt):
    blk = 1024
    return pl.pallas_call(
        _score_body,
        grid=(B // blk,),
        in_specs=[pl.BlockSpec((blk, D), lambda i: (i, 0))] * 3,
        out_specs=pl.BlockSpec((blk, 1), lambda i: (i, 0)),
        out_shape=jax.ShapeDtypeStruct((B, 1), jnp.float32),
    )(xh, rd, xt)


@functools.cache
def _mesh():
    return plsc.VectorSubcoreMesh(core_axis_name="c", subcore_axis_name="s")


def _sc_segsum(xT, ed):
    return _build_sc_segsum()(xT, ed)


@functools.cache
def _build_sc_segsum():
    return functools.partial(
        pl.kernel,
        out_type=jax.ShapeDtypeStruct((R_ENC, D, NP), jnp.float32),
        mesh=_mesh(),
        compiler_params=pltpu.CompilerParams(needs_layout_passes=False),
        scratch_types=[
            pltpu.VMEM((XC * NP,), jnp.float32),
            pltpu.VMEM((XC * NP,), jnp.float32),
            pltpu.VMEM((2 * ECH,), jnp.int32),
            pltpu.VMEM((2 * ECH,), jnp.int32),
            pltpu.SemaphoreType.DMA,
            pltpu.SemaphoreType.DMA,
        ],
    )(_sc_segsum_body)


def _sc_segsum_body(xT_hbm, ed_hbm, s_hbm, x_v, acc_v, e0, e1, sem0, sem1):
    c = lax.axis_index("c")
    s = lax.axis_index("s")
    w = c * NS + s
    ebufs = (e0, e1)
    esems = (sem0, sem1)
    zeros16 = jnp.zeros((16,), jnp.float32)

    for j in range(XC):
        pltpu.sync_copy(xT_hbm.at[XC * w + j], x_v.at[pl.ds(j * NP, NP)])

    def rel_body(rr, carry):
        def zbody(i, cc):
            for q in range(4):
                acc_v[pl.ds(i * 64 + q * 16, 16)] = zeros16
            return cc
        lax.fori_loop(0, XC * NP // 64, zbody, 0)

        pltpu.async_copy(ed_hbm.at[rr, 0], e0, sem0)
        pltpu.async_copy(ed_hbm.at[rr, 1], e1, sem1)

        def run_chunk(buf):
            def gbody(g, cc):
                srcv = buf[pl.ds(g * 16, 16)]
                dstv = buf[pl.ds(ECH + g * 16, 16)]
                for j in range(XC):
                    vals = plsc.load_gather(x_v, [srcv + (j * NP)])
                    plsc.addupdate_scatter(acc_v, [dstv + (j * NP)], vals)
                return cc
            lax.fori_loop(0, ECH // 16, gbody, 0)

        def chunk_pair(k2, cc):
            for bslot in range(2):
                k = k2 * 2 + bslot
                pltpu.make_async_copy(
                    ed_hbm.at[rr, k], ebufs[bslot], esems[bslot]).wait()
                run_chunk(ebufs[bslot])
                pltpu.async_copy(
                    ed_hbm.at[rr, k + 2], ebufs[bslot], esems[bslot])
            return cc

        lax.fori_loop(0, NCH // 2 - 1, chunk_pair, 0)
        for bslot in range(2):
            k = NCH - 2 + bslot
            pltpu.make_async_copy(
                ed_hbm.at[rr, k], ebufs[bslot], esems[bslot]).wait()
            run_chunk(ebufs[bslot])

        for j in range(XC):
            pltpu.sync_copy(acc_v.at[pl.ds(j * NP, NP)], s_hbm.at[rr, XC * w + j])
        return carry

    lax.fori_loop(0, R_ENC, rel_body, 0)


_B_PER_W = B // NW
_DCH = 128
_DEC_CH = _B_PER_W // _DCH


def _sc_decode(x, rel_dec, h2, r2, t2):
    return _build_sc_decode()(x, rel_dec, h2, r2, t2)


@functools.cache
def _build_sc_decode():
    return functools.partial(
        pl.kernel,
        out_type=(jax.ShapeDtypeStruct((B, D), jnp.float32),
                  jax.ShapeDtypeStruct((B, D), jnp.float32),
                  jax.ShapeDtypeStruct((B, D), jnp.float32)),
        mesh=_mesh(),
        scratch_types=[
            pltpu.VMEM((_DCH,), jnp.int32),
            pltpu.VMEM((_DCH, D), jnp.float32),
            pltpu.SemaphoreType.DMA,
        ],
    )(_sc_decode_body)


def _sc_decode_body(x_hbm, rd_hbm, h_hbm, r_hbm, t_hbm,
                    xh_out, rr_out, xt_out, idx_v, buf_v, sem):
    c = lax.axis_index("c")
    s = lax.axis_index("s")
    w = c * NS + s
    for k in range(_DEC_CH):
        row = w * _DEC_CH + k
        base = row * _DCH
        for tab, idx_hbm, out in ((x_hbm, h_hbm, xh_out),
                                  (rd_hbm, r_hbm, rr_out),
                                  (x_hbm, t_hbm, xt_out)):
            pltpu.sync_copy(idx_hbm.at[row], idx_v)
            pltpu.async_copy(tab.at[idx_v], buf_v, sem).wait()
            pltpu.sync_copy(buf_v, out.at[pl.ds(base, _DCH)])


def kernel(entity, self_w, rel_w, ln_g, ln_b, rel_dec, edges_by_rel, h, r, t):
    wcat = jnp.transpose(rel_w, (0, 3, 1, 2)).reshape(L, D, R_ENC * D)

    src = jnp.pad(edges_by_rel[:, 0], ((0, 0), (0, E_PAD - E_PER)),
                  constant_values=0)
    dst = jnp.pad(edges_by_rel[:, 1], ((0, 0), (0, E_PAD - E_PER)),
                  constant_values=N)
    ed = jnp.stack([src.reshape(R_ENC, NCH, ECH),
                    dst.reshape(R_ENC, NCH, ECH)],
                   axis=2).reshape(R_ENC, NCH, 2 * ECH)

    ln_g2 = ln_g.reshape(L, D, 1)
    ln_b2 = ln_b.reshape(L, D, 1)

    xT = jnp.pad(jnp.transpose(entity), ((0, 0), (0, NP - N)))
    for i in range(L):
        s_cat = _sc_segsum(xT, ed).reshape(R_ENC * D, NP)
        xT = _tc_layerT(xT, s_cat, self_w[i], wcat[i], ln_g2[i], ln_b2[i],
                        relu=(i < L - 1))

    x = jnp.transpose(xT)
    h2 = h.reshape(NW * _DEC_CH, _DCH)
    r2 = r.reshape(NW * _DEC_CH, _DCH)
    t2 = t.reshape(NW * _DEC_CH, _DCH)
    xh, rd, xt = _sc_decode(x, rel_dec, h2, r2, t2)
    return _tc_score(xh, rd, xt).reshape(B)


# speedup vs baseline: 1.0073x; 1.0073x over previous
import functools

import jax
import jax.numpy as jnp
from jax import lax
from jax.experimental import pallas as pl
from jax.experimental.pallas import tpu as pltpu
from jax.experimental.pallas import tpu_sc as plsc

N = 10000
D = 128
R_ENC = 8
E_PER = 40000
L = 2
B = 8192
EPS = 1e-5

NW = 32
NS = 16
XC = D // NW
E_PAD = 40960
ECH = 2048
NCH = E_PAD // ECH
NP = 10240
ROW_BLK = 2048


def _lnT_body(xT_ref, s_ref, sw_ref, wcat_ref, g_ref, b_ref, o_ref, *, relu):
    selfT = jnp.dot(sw_ref[...], xT_ref[...],
                    preferred_element_type=jnp.float32)
    msgT = jnp.dot(wcat_ref[...], s_ref[...],
                   preferred_element_type=jnp.float32)
    tot = selfT + msgT
    mu = jnp.mean(tot, axis=0, keepdims=True)
    var = jnp.mean((tot - mu) * (tot - mu), axis=0, keepdims=True)
    hh = (tot - mu) * lax.rsqrt(var + EPS) * g_ref[...] + b_ref[...]
    if relu:
        hh = jnp.maximum(hh, 0.0)
    o_ref[...] = xT_ref[...] + hh


def _tc_layerT(xT, s_cat, sw, wcat, g, b, relu):
    return pl.pallas_call(
        functools.partial(_lnT_body, relu=relu),
        grid=(NP // ROW_BLK,),
        in_specs=[
            pl.BlockSpec((D, ROW_BLK), lambda i: (0, i)),
            pl.BlockSpec((R_ENC * D, ROW_BLK), lambda i: (0, i)),
            pl.BlockSpec((D, D), lambda i: (0, 0)),
            pl.BlockSpec((D, R_ENC * D), lambda i: (0, 0)),
            pl.BlockSpec((D, 1), lambda i: (0, 0)),
            pl.BlockSpec((D, 1), lambda i: (0, 0)),
        ],
        out_specs=pl.BlockSpec((D, ROW_BLK), lambda i: (0, i)),
        out_shape=jax.ShapeDtypeStruct((D, NP), jnp.float32),
    )(xT, s_cat, sw, wcat, g, b)


def _score_body(a_ref, b_ref, c_ref, o_ref):
    o_ref[...] = jnp.sum(a_ref[...] * b_ref[...] * c_ref[...],
                         axis=-1, keepdims=True)


def _tc_score(xh, rd, xt):
    blk = 1024
    return pl.pallas_call(
        _score_body,
        grid=(B // blk,),
        in_specs=[pl.BlockSpec((blk, D), lambda i: (i, 0))] * 3,
        out_specs=pl.BlockSpec((blk, 1), lambda i: (i, 0)),
        out_shape=jax.ShapeDtypeStruct((B, 1), jnp.float32),
    )(xh, rd, xt)


@functools.cache
def _mesh():
    return plsc.VectorSubcoreMesh(core_axis_name="c", subcore_axis_name="s")


def _sc_segsum(xT, ed):
    return _build_sc_segsum()(xT, ed)


@functools.cache
def _build_sc_segsum():
    return functools.partial(
        pl.kernel,
        out_type=jax.ShapeDtypeStruct((R_ENC, D, NP), jnp.float32),
        mesh=_mesh(),
        compiler_params=pltpu.CompilerParams(needs_layout_passes=False),
        scratch_types=(
            [pltpu.VMEM((NP,), jnp.float32)] * XC
            + [pltpu.VMEM((NP,), jnp.float32)] * XC
            + [pltpu.VMEM((2 * ECH,), jnp.int32),
               pltpu.VMEM((2 * ECH,), jnp.int32),
               pltpu.SemaphoreType.DMA,
               pltpu.SemaphoreType.DMA]
        ),
    )(_sc_segsum_body)


def _sc_segsum_body(xT_hbm, ed_hbm, s_hbm,
                    x0, x1, x2, x3, a0, a1, a2, a3, e0, e1, sem0, sem1):
    c = lax.axis_index("c")
    s = lax.axis_index("s")
    w = c * NS + s
    xs = (x0, x1, x2, x3)
    accs = (a0, a1, a2, a3)
    ebufs = (e0, e1)
    esems = (sem0, sem1)
    zeros16 = jnp.zeros((16,), jnp.float32)

    for j in range(XC):
        pltpu.sync_copy(xT_hbm.at[XC * w + j], xs[j])

    def rel_body(rr, carry):
        def zbody(i, cc):
            for j in range(XC):
                for q in range(4):
                    accs[j][pl.ds(i * 64 + q * 16, 16)] = zeros16
            return cc
        lax.fori_loop(0, NP // 64, zbody, 0)

        pltpu.async_copy(ed_hbm.at[rr, 0], e0, sem0)
        pltpu.async_copy(ed_hbm.at[rr, 1], e1, sem1)

        def run_chunk(buf):
            def gbody(g, cc):
                for u in range(4):
                    srcv = buf[pl.ds(g * 64 + u * 16, 16)]
                    dstv = buf[pl.ds(ECH + g * 64 + u * 16, 16)]
                    for j in range(XC):
                        vals = plsc.load_gather(xs[j], [srcv])
                        plsc.addupdate_scatter(accs[j], [dstv], vals)
                return cc
            lax.fori_loop(0, ECH // 64, gbody, 0)

        def chunk_pair(k2, cc):
            for bslot in range(2):
                k = k2 * 2 + bslot
                pltpu.make_async_copy(
                    ed_hbm.at[rr, k], ebufs[bslot], esems[bslot]).wait()
                run_chunk(ebufs[bslot])
                pltpu.async_copy(
                    ed_hbm.at[rr, k + 2], ebufs[bslot], esems[bslot])
            return cc

        lax.fori_loop(0, NCH // 2 - 1, chunk_pair, 0)
        for bslot in range(2):
            k = NCH - 2 + bslot
            pltpu.make_async_copy(
                ed_hbm.at[rr, k], ebufs[bslot], esems[bslot]).wait()
            run_chunk(ebufs[bslot])

        for j in range(XC):
            pltpu.sync_copy(accs[j], s_hbm.at[rr, XC * w + j])
        return carry

    lax.fori_loop(0, R_ENC, rel_body, 0)


_B_PER_W = B // NW
_DCH = 128
_DEC_CH = _B_PER_W // _DCH


def _sc_decode(x, rel_dec, h2, r2, t2):
    return _build_sc_decode()(x, rel_dec, h2, r2, t2)


@functools.cache
def _build_sc_decode():
    return functools.partial(
        pl.kernel,
        out_type=(jax.ShapeDtypeStruct((B, D), jnp.float32),
                  jax.ShapeDtypeStruct((B, D), jnp.float32),
                  jax.ShapeDtypeStruct((B, D), jnp.float32)),
        mesh=_mesh(),
        scratch_types=[
            pltpu.VMEM((_DCH,), jnp.int32),
            pltpu.VMEM((_DCH, D), jnp.float32),
            pltpu.SemaphoreType.DMA,
        ],
    )(_sc_decode_body)


def _sc_decode_body(x_hbm, rd_hbm, h_hbm, r_hbm, t_hbm,
                    xh_out, rr_out, xt_out, idx_v, buf_v, sem):
    c = lax.axis_index("c")
    s = lax.axis_index("s")
    w = c * NS + s
    for k in range(_DEC_CH):
        row = w * _DEC_CH + k
        base = row * _DCH
        for tab, idx_hbm, out in ((x_hbm, h_hbm, xh_out),
                                  (rd_hbm, r_hbm, rr_out),
                                  (x_hbm, t_hbm, xt_out)):
            pltpu.sync_copy(idx_hbm.at[row], idx_v)
            pltpu.async_copy(tab.at[idx_v], buf_v, sem).wait()
            pltpu.sync_copy(buf_v, out.at[pl.ds(base, _DCH)])


def kernel(entity, self_w, rel_w, ln_g, ln_b, rel_dec, edges_by_rel, h, r, t):
    wcat = jnp.transpose(rel_w, (0, 3, 1, 2)).reshape(L, D, R_ENC * D)

    src = jnp.pad(edges_by_rel[:, 0], ((0, 0), (0, E_PAD - E_PER)),
                  constant_values=0)
    dst = jnp.pad(edges_by_rel[:, 1], ((0, 0), (0, E_PAD - E_PER)),
                  constant_values=N)
    ed = jnp.stack([src.reshape(R_ENC, NCH, ECH),
                    dst.reshape(R_ENC, NCH, ECH)],
                   axis=2).reshape(R_ENC, NCH, 2 * ECH)

    ln_g2 = ln_g.reshape(L, D, 1)
    ln_b2 = ln_b.reshape(L, D, 1)

    xT = jnp.pad(jnp.transpose(entity), ((0, 0), (0, NP - N)))
    for i in range(L):
        s_cat = _sc_segsum(xT, ed).reshape(R_ENC * D, NP)
        xT = _tc_layerT(xT, s_cat, self_w[i], wcat[i], ln_g2[i], ln_b2[i],
                        relu=(i < L - 1))

    x = jnp.transpose(xT)
    h2 = h.reshape(NW * _DEC_CH, _DCH)
    r2 = r.reshape(NW * _DEC_CH, _DCH)
    t2 = t.reshape(NW * _DEC_CH, _DCH)
    xh, rd, xt = _sc_decode(x, rel_dec, h2, r2, t2)
    return _tc_score(xh, rd, xt).reshape(B)


# speedup vs baseline: 1.2645x; 1.2553x over previous
import functools

import jax
import jax.numpy as jnp
from jax import lax
from jax.experimental import pallas as pl
from jax.experimental.pallas import tpu as pltpu
from jax.experimental.pallas import tpu_sc as plsc

N = 10000
D = 128
R_ENC = 8
E_PER = 40000
L = 2
B = 8192
EPS = 1e-5

NW = 32
NS = 16
E_PAD = 40960
CH = 80
CHUNKS_PER_W = (R_ENC * E_PAD) // (NW * CH)
ACC_ROWS = 10240
STRIPE = ACC_ROWS // NS
ROW_BLK = 1000


def _mm_body(x_ref, w_ref, z_ref):
    z_ref[...] = jnp.dot(x_ref[...], w_ref[...],
                         preferred_element_type=jnp.float32)


def _tc_matmul(x, w_cat):
    return pl.pallas_call(
        _mm_body,
        grid=(N // ROW_BLK,),
        in_specs=[
            pl.BlockSpec((ROW_BLK, D), lambda i: (i, 0)),
            pl.BlockSpec((D, (R_ENC + 1) * D), lambda i: (0, 0)),
        ],
        out_specs=pl.BlockSpec((ROW_BLK, (R_ENC + 1) * D), lambda i: (i, 0)),
        out_shape=jax.ShapeDtypeStruct((N, (R_ENC + 1) * D), jnp.float32),
    )(x, w_cat)


def _ln_body(zs_ref, m_ref, x_ref, g_ref, b_ref, o_ref, *, relu):
    tot = zs_ref[...] + m_ref[0] + m_ref[1]
    mu = jnp.mean(tot, axis=-1, keepdims=True)
    var = jnp.mean((tot - mu) * (tot - mu), axis=-1, keepdims=True)
    hh = (tot - mu) * lax.rsqrt(var + EPS) * g_ref[...] + b_ref[...]
    if relu:
        hh = jnp.maximum(hh, 0.0)
    o_ref[...] = x_ref[...] + hh


def _tc_layernorm(z, msg, x, g, b, relu):
    return pl.pallas_call(
        functools.partial(_ln_body, relu=relu),
        grid=(N // ROW_BLK,),
        in_specs=[
            pl.BlockSpec((ROW_BLK, D), lambda i: (i, 0)),
            pl.BlockSpec((2, ROW_BLK, D), lambda i: (0, i, 0)),
            pl.BlockSpec((ROW_BLK, D), lambda i: (i, 0)),
            pl.BlockSpec((1, D), lambda i: (0, 0)),
            pl.BlockSpec((1, D), lambda i: (0, 0)),
        ],
        out_specs=pl.BlockSpec((ROW_BLK, D), lambda i: (i, 0)),
        out_shape=jax.ShapeDtypeStruct((N, D), jnp.float32),
    )(z, msg, x, g, b)


def _score_body(a_ref, b_ref, c_ref, o_ref):
    o_ref[...] = jnp.sum(a_ref[...] * b_ref[...] * c_ref[...],
                         axis=-1, keepdims=True)


def _tc_score(xh, rd, xt):
    blk = 1024
    return pl.pallas_call(
        _score_body,
        grid=(B // blk,),
        in_specs=[pl.BlockSpec((blk, D), lambda i: (i, 0))] * 3,
        out_specs=pl.BlockSpec((blk, 1), lambda i: (i, 0)),
        out_shape=jax.ShapeDtypeStruct((B, 1), jnp.float32),
    )(xh, rd, xt)


@functools.cache
def _mesh():
    return plsc.VectorSubcoreMesh(core_axis_name="c", subcore_axis_name="s")


def _sc_aggregate(table, srcw, dstw, zeros):
    return _build_sc_aggregate()(table, srcw, dstw, zeros)


@functools.cache
def _build_sc_aggregate():
    return functools.partial(
        pl.kernel,
        out_type=jax.ShapeDtypeStruct((2, N, D), jnp.float32),
        mesh=_mesh(),
        scratch_types=[
            pltpu.VMEM((_PHASE_CH, CH), jnp.int32),
            pltpu.VMEM((_PHASE_CH, CH), jnp.int32),
            pltpu.VMEM((CH, D), jnp.float32),
            pltpu.VMEM((CH, D), jnp.float32),
            pltpu.VMEM((CH, D), jnp.float32),
            pltpu.VMEM((CH, D), jnp.float32),
            pltpu.VMEM_SHARED((ACC_ROWS, D), jnp.float32),
            pltpu.SemaphoreType.DMA,
            pltpu.SemaphoreType.DMA,
            pltpu.SemaphoreType.DMA,
            pltpu.SemaphoreType.DMA,
        ],
    )(_sc_aggregate_body)


_NBUF = 4
_PHASES = 4
_PHASE_CH = CHUNKS_PER_W // _PHASES


def _sc_aggregate_body(table_hbm, srcw_hbm, dstw_hbm, zeros_hbm, out_hbm,
                       src_v, dst_v, b0, b1, b2, b3, acc_sh, g0, g1, g2, g3):
    c = lax.axis_index("c")
    s = lax.axis_index("s")
    w = c * NS + s
    bufs = (b0, b1, b2, b3)
    gsem = (g0, g1, g2, g3)
    pltpu.sync_copy(zeros_hbm, acc_sh.at[pl.ds(s * STRIPE, STRIPE)])
    plsc.subcore_barrier()

    for ph in range(_PHASES):
        pltpu.sync_copy(srcw_hbm.at[w, pl.ds(ph * _PHASE_CH, _PHASE_CH)], src_v)
        pltpu.sync_copy(dstw_hbm.at[w, pl.ds(ph * _PHASE_CH, _PHASE_CH)], dst_v)
        for b in range(_NBUF):
            pltpu.async_copy(table_hbm.at[src_v.at[b]], bufs[b], gsem[b])

        def body(i, carry):
            for b in range(_NBUF):
                ch = i * _NBUF + b
                pltpu.make_async_copy(
                    table_hbm.at[src_v.at[ch]], bufs[b], gsem[b]).wait()
                pltpu.sync_copy(bufs[b], acc_sh.at[dst_v.at[ch]], add=True)
                pltpu.async_copy(
                    table_hbm.at[src_v.at[ch + _NBUF]], bufs[b], gsem[b])
            return carry

        lax.fori_loop(0, _PHASE_CH // _NBUF - 1, body, 0)
        for b in range(_NBUF):
            ch = _PHASE_CH - _NBUF + b
            pltpu.make_async_copy(
                table_hbm.at[src_v.at[ch]], bufs[b], gsem[b]).wait()
            pltpu.sync_copy(bufs[b], acc_sh.at[dst_v.at[ch]], add=True)
    plsc.subcore_barrier()
    @pl.when(s < NS - 1)
    def _():
        pltpu.sync_copy(acc_sh.at[pl.ds(s * STRIPE, STRIPE)],
                        out_hbm.at[c, pl.ds(s * STRIPE, STRIPE)])

    @pl.when(s == NS - 1)
    def _():
        pltpu.sync_copy(acc_sh.at[pl.ds((NS - 1) * STRIPE, N - (NS - 1) * STRIPE)],
                        out_hbm.at[c, pl.ds((NS - 1) * STRIPE, N - (NS - 1) * STRIPE)])


_B_PER_W = B // NW
_DCH = 128
_DEC_CH = _B_PER_W // _DCH


def _sc_decode(x, rel_dec, h2, r2, t2):
    return _build_sc_decode()(x, rel_dec, h2, r2, t2)


@functools.cache
def _build_sc_decode():
    return functools.partial(
        pl.kernel,
        out_type=(jax.ShapeDtypeStruct((B, D), jnp.float32),
                  jax.ShapeDtypeStruct((B, D), jnp.float32),
                  jax.ShapeDtypeStruct((B, D), jnp.float32)),
        mesh=_mesh(),
        scratch_types=[
            pltpu.VMEM((_DCH,), jnp.int32),
            pltpu.VMEM((_DCH, D), jnp.float32),
            pltpu.SemaphoreType.DMA,
        ],
    )(_sc_decode_body)


def _sc_decode_body(x_hbm, rd_hbm, h_hbm, r_hbm, t_hbm,
                    xh_out, rr_out, xt_out, idx_v, buf_v, sem):
    c = lax.axis_index("c")
    s = lax.axis_index("s")
    w = c * NS + s
    for k in range(_DEC_CH):
        row = w * _DEC_CH + k
        base = row * _DCH
        for tab, idx_hbm, out in ((x_hbm, h_hbm, xh_out),
                                  (rd_hbm, r_hbm, rr_out),
                                  (x_hbm, t_hbm, xt_out)):
            pltpu.sync_copy(idx_hbm.at[row], idx_v)
            pltpu.async_copy(tab.at[idx_v], buf_v, sem).wait()
            pltpu.sync_copy(buf_v, out.at[pl.ds(base, _DCH)])


def kernel(entity, self_w, rel_w, ln_g, ln_b, rel_dec, edges_by_rel, h, r, t):
    f32 = jnp.float32
    blocks = jnp.concatenate(
        [jnp.transpose(self_w, (0, 2, 1))[:, None], rel_w], axis=1)
    w_cat = jnp.transpose(blocks, (0, 2, 1, 3)).reshape(L, D, (R_ENC + 1) * D)

    src = edges_by_rel[:, 0]
    dst = edges_by_rel[:, 1]
    rel_off = 1 + jnp.arange(R_ENC, dtype=jnp.int32)[:, None]
    srcf = src * (R_ENC + 1) + rel_off
    srcf = jnp.pad(srcf, ((0, 0), (0, E_PAD - E_PER)), constant_values=0)
    dstf = jnp.pad(dst, ((0, 0), (0, E_PAD - E_PER)), constant_values=N)
    srcw = srcf.reshape(NW, CHUNKS_PER_W, CH)
    dstw = dstf.reshape(NW, CHUNKS_PER_W, CH)
    zeros = jnp.zeros((STRIPE, D), f32)

    ln_g2 = ln_g.reshape(L, 1, D)
    ln_b2 = ln_b.reshape(L, 1, D)

    x = entity
    for i in range(L):
        z = _tc_matmul(x, w_cat[i])
        zflat = z.reshape((R_ENC + 1) * N, D)
        msg = _sc_aggregate(zflat, srcw, dstw, zeros)
        x = _tc_layernorm(z, msg, x, ln_g2[i], ln_b2[i], relu=(i < L - 1))

    h2 = h.reshape(NW * _DEC_CH, _DCH)
    r2 = r.reshape(NW * _DEC_CH, _DCH)
    t2 = t.reshape(NW * _DEC_CH, _DCH)
    xh, rd, xt = _sc_decode(x, rel_dec, h2, r2, t2)
    return _tc_score(xh, rd, xt).reshape(B)
